# baseline (speedup 1.0000x reference)
import functools

import jax
import jax.numpy as jnp
import numpy as np
from jax import lax
from jax.experimental import pallas as pl
from jax.experimental.pallas import tpu as pltpu
from jax.experimental.pallas import tpu_sc as plsc

B, S, D = 16, 4096, 1024
ROWS = B * S

_info = plsc.get_sparse_core_info()
NC, NS = _info.num_cores, _info.num_subcores
NW = NC * NS
RPW = ROWS // NW
K = 32
NCHUNK = RPW // K


_FLAT_IDX = None


def _rotl32(x, r):
    return ((x << np.uint32(r)) | (x >> np.uint32(32 - r))).astype(np.uint32)


def _threefry_bits(k0, k1, n):
    x0 = np.zeros(n, dtype=np.uint32)
    x1 = np.arange(n, dtype=np.uint32)
    ks0 = np.uint32(k0)
    ks1 = np.uint32(k1)
    ks2 = np.uint32(ks0 ^ ks1 ^ np.uint32(0x1BD11BDA))
    rots = [(13, 15, 26, 6), (17, 29, 16, 24)]
    inject = [(ks1, ks2), (ks2, ks0), (ks0, ks1), (ks1, ks2), (ks2, ks0)]
    x0 = (x0 + ks0).astype(np.uint32)
    x1 = (x1 + ks1).astype(np.uint32)
    for blk in range(5):
        for r in rots[blk % 2]:
            x0 = (x0 + x1).astype(np.uint32)
            x1 = _rotl32(x1, r)
            x1 = (x1 ^ x0).astype(np.uint32)
        a, b = inject[blk]
        x0 = (x0 + a).astype(np.uint32)
        x1 = (x1 + b + np.uint32(blk + 1)).astype(np.uint32)
    return (x0 ^ x1).astype(np.uint32)


def _flat_indices() -> np.ndarray:
    global _FLAT_IDX
    if _FLAT_IDX is None:
        bits = _threefry_bits(0, 42, B * S)
        u = ((bits >> np.uint32(9)) | np.uint32(0x3F800000)).view(np.float32)
        rand = np.maximum(np.float32(0.0), u - np.float32(1.0)).reshape(B, S)
        perm = np.argsort(rand, axis=1, kind="stable").astype(np.int32)
        _FLAT_IDX = (perm + (np.arange(B, dtype=np.int32) * S)[:, None]).reshape(-1)
    return _FLAT_IDX


_mesh = plsc.VectorSubcoreMesh(core_axis_name="c", subcore_axis_name="s")


NBUF = 3


@functools.partial(
    pl.kernel,
    out_type=jax.ShapeDtypeStruct((ROWS, D), jnp.float32),
    mesh=_mesh,
    scratch_types=[
        pltpu.VMEM((RPW,), jnp.int32),
        pltpu.VMEM((K, D), jnp.float32),
        pltpu.VMEM((K, D), jnp.float32),
        pltpu.VMEM((K, D), jnp.float32),
        pltpu.SemaphoreType.DMA,
        pltpu.SemaphoreType.DMA,
        pltpu.SemaphoreType.DMA,
        pltpu.SemaphoreType.DMA,
        pltpu.SemaphoreType.DMA,
        pltpu.SemaphoreType.DMA,
    ],
)
def _sc_permute_rows(x_hbm, gidx_hbm, out_hbm, idx_v,
                     buf0, buf1, buf2, isem0, isem1, isem2,
                     osem0, osem1, osem2):
    wid = lax.axis_index("s") * NC + lax.axis_index("c")
    base = wid * RPW
    pltpu.sync_copy(gidx_hbm.at[pl.ds(base, RPW)], idx_v)

    bufs = (buf0, buf1, buf2)
    isems = (isem0, isem1, isem2)
    osems = (osem0, osem1, osem2)

    def start_in(g, b):
        pltpu.async_copy(x_hbm.at[idx_v.at[pl.ds(g * K, K)]], bufs[b], isems[b])

    def wait_in(b):
        pltpu.make_async_copy(x_hbm.at[pl.ds(0, K)], bufs[b], isems[b]).wait()

    def start_out(g, b):
        pltpu.async_copy(bufs[b], out_hbm.at[pl.ds(base + g * K, K)], osems[b])

    def wait_out(b):
        pltpu.make_async_copy(x_hbm.at[pl.ds(0, K)], bufs[b], osems[b]).wait()

    def step(g, b, bn, lookahead=True):
        wait_in(b)
        start_out(g, b)
        if lookahead:
            wait_out(bn)
            start_in(g + 2, bn)

    start_in(0, 0)
    start_in(1, 1)

    wait_in(0)
    start_out(0, 0)
    start_in(2, 2)
    step(1, 1, 0)
    step(2, 2, 1)

    def body(g3, carry):
        g0 = 3 + g3 * 3
        for db in range(3):
            step(g0 + db, db, (db + 2) % 3)
        return carry

    assert (NCHUNK - 7) % 3 == 0
    lax.fori_loop(0, (NCHUNK - 7) // 3, body, 0)
    step(NCHUNK - 4, (NCHUNK - 4) % 3, (NCHUNK - 2) % 3)
    step(NCHUNK - 3, (NCHUNK - 3) % 3, (NCHUNK - 1) % 3)
    step(NCHUNK - 2, (NCHUNK - 2) % 3, 0, lookahead=False)
    step(NCHUNK - 1, (NCHUNK - 1) % 3, 0, lookahead=False)
    for b in range(NBUF):
        wait_out(b)


def kernel(x):
    gidx = jnp.asarray(_flat_indices())
    z = _sc_permute_rows(x.reshape(ROWS, D), gidx)
    return z.reshape(B, S, D), jnp.zeros((B,), jnp.float32)


# speedup vs baseline: 1.0062x; 1.0062x over previous
import functools

import jax
import jax.numpy as jnp
import numpy as np
from jax import lax
from jax.experimental import pallas as pl
from jax.experimental.pallas import tpu as pltpu
from jax.experimental.pallas import tpu_sc as plsc

B, S, D = 16, 4096, 1024
ROWS = B * S

_info = plsc.get_sparse_core_info()
NC, NS = _info.num_cores, _info.num_subcores
NW = NC * NS
RPW = ROWS // NW
K = 32
NCHUNK = RPW // K


_FLAT_IDX = None


def _rotl32(x, r):
    return ((x << np.uint32(r)) | (x >> np.uint32(32 - r))).astype(np.uint32)


def _threefry_bits(k0, k1, n):
    x0 = np.zeros(n, dtype=np.uint32)
    x1 = np.arange(n, dtype=np.uint32)
    ks0 = np.uint32(k0)
    ks1 = np.uint32(k1)
    ks2 = np.uint32(ks0 ^ ks1 ^ np.uint32(0x1BD11BDA))
    rots = [(13, 15, 26, 6), (17, 29, 16, 24)]
    inject = [(ks1, ks2), (ks2, ks0), (ks0, ks1), (ks1, ks2), (ks2, ks0)]
    x0 = (x0 + ks0).astype(np.uint32)
    x1 = (x1 + ks1).astype(np.uint32)
    for blk in range(5):
        for r in rots[blk % 2]:
            x0 = (x0 + x1).astype(np.uint32)
            x1 = _rotl32(x1, r)
            x1 = (x1 ^ x0).astype(np.uint32)
        a, b = inject[blk]
        x0 = (x0 + a).astype(np.uint32)
        x1 = (x1 + b + np.uint32(blk + 1)).astype(np.uint32)
    return (x0 ^ x1).astype(np.uint32)


def _flat_indices() -> np.ndarray:
    global _FLAT_IDX
    if _FLAT_IDX is None:
        bits = _threefry_bits(0, 42, B * S)
        u = ((bits >> np.uint32(9)) | np.uint32(0x3F800000)).view(np.float32)
        rand = np.maximum(np.float32(0.0), u - np.float32(1.0)).reshape(B, S)
        perm = np.argsort(rand, axis=1, kind="stable").astype(np.int32)
        _FLAT_IDX = (perm + (np.arange(B, dtype=np.int32) * S)[:, None]).reshape(-1)
    return _FLAT_IDX


_mesh = plsc.VectorSubcoreMesh(core_axis_name="c", subcore_axis_name="s")


@functools.partial(
    pl.kernel,
    out_type=jax.ShapeDtypeStruct((ROWS, D), jnp.float32),
    mesh=_mesh,
    scratch_types=[
        pltpu.VMEM((RPW,), jnp.int32),
        pltpu.VMEM((K, D), jnp.float32),
        pltpu.VMEM((K, D), jnp.float32),
        pltpu.SemaphoreType.DMA,
        pltpu.SemaphoreType.DMA,
    ],
)
def _sc_permute_rows(x_hbm, gidx_hbm, out_hbm, idx_v, buf0, buf1, sem0, sem1):
    wid = lax.axis_index("s") * NC + lax.axis_index("c")
    base = wid * RPW
    pltpu.sync_copy(gidx_hbm.at[pl.ds(base, RPW)], idx_v)

    bufs = (buf0, buf1)
    sems = (sem0, sem1)

    def start(g, b):
        pltpu.async_copy(x_hbm.at[idx_v.at[pl.ds(g * K, K)]], bufs[b], sems[b])

    def wait(b):
        pltpu.make_async_copy(x_hbm.at[pl.ds(0, K)], bufs[b], sems[b]).wait()

    def put(g, b):
        pltpu.sync_copy(bufs[b], out_hbm.at[pl.ds(base + g * K, K)])

    start(0, 0)
    start(1, 1)

    def body(g2, carry):
        for b in range(2):
            g = g2 * 2 + b
            wait(b)
            put(g, b)
            start(g + 2, b)
        return carry

    lax.fori_loop(0, NCHUNK // 2 - 1, body, 0)
    for b in range(2):
        wait(b)
        put(NCHUNK - 2 + b, b)


def kernel(x):
    gidx = jnp.asarray(_flat_indices())
    z = _sc_permute_rows(x.reshape(ROWS, D), gidx)
    return z.reshape(B, S, D), jnp.zeros((B,), jnp.float32)
